# baseline (speedup 1.0000x reference)
import functools

import jax
import jax.numpy as jnp
from jax import lax
from jax.experimental import pallas as pl
from jax.experimental.pallas import tpu as pltpu
from jax.experimental.pallas import tpu_sc as plsc

T, D, H, O, E, TOPK = 2048, 1024, 1024, 1024, 8, 2
GH = 256
K2 = H // 2
BT = 256
NA = T * TOPK
NBLK = NA // BT + (E - 1)
NPAD = NBLK * BT
NW = 32
LNEPS = 1e-5


def _ln(x, g, b):
    mu = jnp.mean(x, axis=-1, keepdims=True)
    var = jnp.mean((x - mu) ** 2, axis=-1, keepdims=True)
    return (x - mu) * jax.lax.rsqrt(var + LNEPS) * g + b


def _gating_body(x_ref, wgin_ref, bgin_ref, ggln1_ref, bgln1_ref, wgh1_ref,
                 bgh1_ref, ggln2_ref, bgln2_ref, wgh2_ref, bgh2_ref,
                 wgout_ref, bgout_ref, temp_ref,
                 probs_ref, wa_ref, wb_ref, pos_ref, counts_ref, lb_ref):
    x = x_ref[...]
    h0 = jax.nn.relu(
        lax.dot_general(x, wgin_ref[...], (((1,), (1,)), ((), ())),
                        preferred_element_type=jnp.float32) + bgin_ref[...])
    h1 = jax.nn.relu(_ln(h0, ggln1_ref[...], bgln1_ref[...]))
    h1 = lax.dot_general(h1, wgh1_ref[...], (((1,), (1,)), ((), ())),
                         preferred_element_type=jnp.float32) + bgh1_ref[...]
    h1 = h1 + h0
    h2 = jax.nn.relu(_ln(h1, ggln2_ref[...], bgln2_ref[...]))
    h2 = lax.dot_general(h2, wgh2_ref[...], (((1,), (1,)), ((), ())),
                         preferred_element_type=jnp.float32) + bgh2_ref[...]
    logits = lax.dot_general(h2, wgout_ref[...], (((1,), (1,)), ((), ())),
                             preferred_element_type=jnp.float32) + bgout_ref[...]
    temp = jnp.clip(temp_ref[0, 0], 0.5, 5.0)
    logits = logits / temp
    logits = logits - jnp.max(logits, axis=1, keepdims=True)
    ex = jnp.exp(logits)
    p = ex / jnp.sum(ex, axis=1, keepdims=True)
    probs_ref[...] = p

    usage = jnp.mean(p, axis=0, keepdims=True)
    lb_ref[...] = jnp.sum((usage - 1.0 / E) ** 2).reshape(1, 1) * 0.05

    iota = lax.broadcasted_iota(jnp.int32, (T, E), 1).astype(jnp.float32)
    m1 = jnp.max(p, axis=1, keepdims=True)
    i1 = jnp.min(jnp.where(p == m1, iota, float(E)), axis=1, keepdims=True)
    pm = jnp.where(iota == i1, -jnp.inf, p)
    m2 = jnp.max(pm, axis=1, keepdims=True)
    i2 = jnp.min(jnp.where(pm == m2, iota, float(E)), axis=1, keepdims=True)
    ssum = m1 + m2
    wa_ref[...] = jnp.broadcast_to(m1 / ssum, (T, 16))
    wb_ref[...] = jnp.broadcast_to(m2 / ssum, (T, 16))

    oh = jnp.concatenate([(iota == i1).astype(jnp.float32),
                          (iota == i2).astype(jnp.float32)], axis=0)

    C, Rr = 8, NA // 8
    tri = (lax.broadcasted_iota(jnp.int32, (Rr, Rr), 0)
           >= lax.broadcasted_iota(jnp.int32, (Rr, Rr), 1)).astype(jnp.float32)
    offs = jnp.zeros((1, E), jnp.float32)
    pos_rows = []
    for c in range(C):
        ohc = oh[c * Rr:(c + 1) * Rr]
        within = lax.dot_general(tri, ohc, (((1,), (0,)), ((), ())),
                                 preferred_element_type=jnp.float32)
        rank_c = (jnp.sum(within * ohc, axis=1, keepdims=True) - 1.0
                  + jnp.sum(ohc * offs, axis=1, keepdims=True))
        pos_rows.append((rank_c, ohc))
        offs = offs + within[Rr - 1:Rr, :]
    counts = offs
    counts_ref[...] = counts.astype(jnp.int32)

    sizes_al = jnp.floor((counts + (BT - 1)) / BT) * BT
    tri8 = (lax.broadcasted_iota(jnp.int32, (E, E), 0)
            < lax.broadcasted_iota(jnp.int32, (E, E), 1)).astype(jnp.float32)
    starts = lax.dot_general(sizes_al, tri8, (((1,), (0,)), ((), ())),
                             preferred_element_type=jnp.float32)
    for c in range(C):
        rank_c, ohc = pos_rows[c]
        pos_c = rank_c + jnp.sum(ohc * starts, axis=1, keepdims=True)
        pos_ref[c:c + 1, :] = pos_c.reshape(1, Rr).astype(jnp.int32)


def _run_gating(x, W_gin, b_gin, g_gln1, b_gln1, W_gh1, b_gh1, g_gln2, b_gln2,
                W_gh2, b_gh2, W_gout, b_gout, temperature):
    out_shapes = (
        jax.ShapeDtypeStruct((T, E), jnp.float32),
        jax.ShapeDtypeStruct((T, 16), jnp.float32),
        jax.ShapeDtypeStruct((T, 16), jnp.float32),
        jax.ShapeDtypeStruct((8, NA // 8), jnp.int32),
        jax.ShapeDtypeStruct((1, E), jnp.int32),
        jax.ShapeDtypeStruct((1, 1), jnp.float32),
    )
    return pl.pallas_call(
        _gating_body,
        out_shape=out_shapes,
    )(x, W_gin, b_gin.reshape(1, GH), g_gln1.reshape(1, GH),
      b_gln1.reshape(1, GH), W_gh1, b_gh1.reshape(1, GH),
      g_gln2.reshape(1, GH), b_gln2.reshape(1, GH), W_gh2,
      b_gh2.reshape(1, 128), W_gout, b_gout.reshape(1, E),
      temperature.reshape(1, 1))


def _dispatch_body(x_hbm, pa_hbm, pb_hbm, xs_hbm, rows_v, ia_v, ib_v,
                   sema, semb):
    wid = lax.axis_index("s") * 2 + lax.axis_index("c")
    per = T // NW
    for it in range(per // 32):
        tb = wid * per + it * 32
        pltpu.sync_copy(pa_hbm.at[pl.ds(tb, 32)], ia_v)
        pltpu.sync_copy(pb_hbm.at[pl.ds(tb, 32)], ib_v)
        pltpu.sync_copy(x_hbm.at[pl.ds(tb, 32)], rows_v)
        ca = pltpu.async_copy(rows_v, xs_hbm.at[ia_v], sema)
        cb = pltpu.async_copy(rows_v, xs_hbm.at[ib_v], semb)
        ca.wait()
        cb.wait()


def _run_dispatch(x, pos_a, pos_b):
    mesh = plsc.VectorSubcoreMesh(core_axis_name="c", subcore_axis_name="s")
    f = pl.kernel(
        _dispatch_body,
        out_type=jax.ShapeDtypeStruct((NPAD, D), jnp.float32),
        mesh=mesh,
        scratch_types=[
            pltpu.VMEM((32, D), jnp.float32),
            pltpu.VMEM((32,), jnp.int32),
            pltpu.VMEM((32,), jnp.int32),
            pltpu.SemaphoreType.DMA,
            pltpu.SemaphoreType.DMA,
        ],
    )
    return f(x, pos_a, pos_b)


def _expert_body(be_ref, xs_ref, win_ref, bin_ref, gln1_ref, bln1_ref,
                 wh1_ref, bh1_ref, gln2_ref, bln2_ref, wh2_ref, bh2_ref,
                 wout_ref, bout_ref, y_ref):
    bf = jnp.bfloat16
    xb = xs_ref[...].astype(bf)
    h0 = jax.nn.relu(
        lax.dot_general(xb, win_ref[0], (((1,), (1,)), ((), ())),
                        preferred_element_type=jnp.float32) + bin_ref[0])
    t1 = jax.nn.relu(_ln(h0, gln1_ref[0], bln1_ref[0]))
    t1 = lax.dot_general(t1.astype(bf), wh1_ref[0], (((1,), (1,)), ((), ())),
                         preferred_element_type=jnp.float32) + bh1_ref[0]
    t1 = t1 + h0
    t2 = _ln(t1, gln2_ref[0], bln2_ref[0])
    t2 = t2 * jax.nn.sigmoid(t2)
    t2 = lax.dot_general(t2.astype(bf), wh2_ref[0], (((1,), (1,)), ((), ())),
                         preferred_element_type=jnp.float32) + bh2_ref[0]
    y_ref[...] = lax.dot_general(t2.astype(bf), wout_ref[0],
                                 (((1,), (1,)), ((), ())),
                                 preferred_element_type=jnp.float32) + bout_ref[0]


def _run_experts(x_sorted, blk_expert, We_in, be_in, ge_ln1, be_ln1, We_h1,
                 be_h1, ge_ln2, be_ln2, We_h2, be_h2, We_out, be_out):
    def im_x(b, be):
        return (b, 0)

    def im_e3(b, be):
        return (be[b], 0, 0)

    def im_e2(b, be):
        return (be[b], 0)

    grid_spec = pltpu.PrefetchScalarGridSpec(
        num_scalar_prefetch=1,
        grid=(NBLK,),
        in_specs=[
            pl.BlockSpec((BT, D), im_x),
            pl.BlockSpec((1, H, D), im_e3),
            pl.BlockSpec((1, 1, H), im_e3),
            pl.BlockSpec((1, 1, H), im_e3),
            pl.BlockSpec((1, 1, H), im_e3),
            pl.BlockSpec((1, H, H), im_e3),
            pl.BlockSpec((1, 1, H), im_e3),
            pl.BlockSpec((1, 1, H), im_e3),
            pl.BlockSpec((1, 1, H), im_e3),
            pl.BlockSpec((1, K2, H), im_e3),
            pl.BlockSpec((1, 1, K2), im_e3),
            pl.BlockSpec((1, O, K2), im_e3),
            pl.BlockSpec((1, 1, O), im_e3),
        ],
        out_specs=pl.BlockSpec((BT, O), im_x),
    )
    return pl.pallas_call(
        _expert_body,
        grid_spec=grid_spec,
        out_shape=jax.ShapeDtypeStruct((NPAD, O), jnp.float32),
        compiler_params=pltpu.CompilerParams(
            dimension_semantics=("arbitrary",)),
    )(blk_expert, x_sorted, We_in.astype(jnp.bfloat16),
      be_in.reshape(E, 1, H), ge_ln1.reshape(E, 1, H),
      be_ln1.reshape(E, 1, H), We_h1.astype(jnp.bfloat16),
      be_h1.reshape(E, 1, H), ge_ln2.reshape(E, 1, H),
      be_ln2.reshape(E, 1, H), We_h2.astype(jnp.bfloat16),
      be_h2.reshape(E, 1, K2), We_out.astype(jnp.bfloat16),
      be_out.reshape(E, 1, O))


def _combine_body(y_hbm, pa_hbm, pb_hbm, wa_hbm, wb_hbm, fin_hbm,
                  bufa, bufb, ia_v, ib_v, wa_v, wb_v, sema, semb):
    wid = lax.axis_index("s") * 2 + lax.axis_index("c")
    per = T // NW
    for it in range(per // 32):
        tb = wid * per + it * 32
        pltpu.sync_copy(pa_hbm.at[pl.ds(tb, 32)], ia_v)
        pltpu.sync_copy(pb_hbm.at[pl.ds(tb, 32)], ib_v)
        ca = pltpu.async_copy(y_hbm.at[ia_v], bufa, sema)
        cb = pltpu.async_copy(y_hbm.at[ib_v], bufb, semb)
        pltpu.sync_copy(wa_hbm.at[pl.ds(tb, 32)], wa_v)
        pltpu.sync_copy(wb_hbm.at[pl.ds(tb, 32)], wb_v)
        ca.wait()
        cb.wait()

        def body(i, carry):
            va = wa_v[i]
            vb = wb_v[i]
            for j in range(O // 16):
                sl = pl.ds(j * 16, 16)
                bufa[i, sl] = va * bufa[i, sl] + vb * bufb[i, sl]
            return carry

        lax.fori_loop(0, 32, body, 0)
        pltpu.sync_copy(bufa, fin_hbm.at[pl.ds(tb, 32)])


def _run_combine(y, pos_a, pos_b, w_a, w_b):
    mesh = plsc.VectorSubcoreMesh(core_axis_name="c", subcore_axis_name="s")
    f = pl.kernel(
        _combine_body,
        out_type=jax.ShapeDtypeStruct((T, O), jnp.float32),
        mesh=mesh,
        scratch_types=[
            pltpu.VMEM((32, O), jnp.float32),
            pltpu.VMEM((32, O), jnp.float32),
            pltpu.VMEM((32,), jnp.int32),
            pltpu.VMEM((32,), jnp.int32),
            pltpu.VMEM((32, 16), jnp.float32),
            pltpu.VMEM((32, 16), jnp.float32),
            pltpu.SemaphoreType.DMA,
            pltpu.SemaphoreType.DMA,
        ],
    )
    return f(y, pos_a, pos_b, w_a, w_b)


def kernel(x, W_gin, b_gin, g_gln1, b_gln1, W_gh1, b_gh1, g_gln2, b_gln2,
           W_gh2, b_gh2, W_gout, b_gout, temperature, We_in, be_in, ge_ln1,
           be_ln1, We_h1, be_h1, ge_ln2, be_ln2, We_h2, be_h2, We_out,
           be_out):
    gate_probs, w_a, w_b, pos8, counts, lb = _run_gating(
        x, W_gin, b_gin, g_gln1, b_gln1, W_gh1, b_gh1, g_gln2, b_gln2,
        W_gh2, b_gh2, W_gout, b_gout, temperature)

    pos_flat = pos8.reshape(NA)
    pos_a = pos_flat[:T]
    pos_b = pos_flat[T:]

    counts_i = counts.reshape(E)
    sizes_al = ((counts_i + (BT - 1)) // BT) * BT
    starts = jnp.concatenate(
        [jnp.zeros((1,), jnp.int32), jnp.cumsum(sizes_al)[:-1]]).astype(jnp.int32)
    b_lo = jnp.arange(NBLK, dtype=jnp.int32) * BT
    in_blk = (b_lo[:, None] >= starts[None, :]) & \
             (b_lo[:, None] < (starts + sizes_al)[None, :])
    blk_expert = jnp.sum(
        in_blk * jnp.arange(E, dtype=jnp.int32)[None, :], axis=1).astype(jnp.int32)

    x_sorted = _run_dispatch(x, pos_a, pos_b)
    y = _run_experts(x_sorted, blk_expert, We_in, be_in, ge_ln1, be_ln1,
                     We_h1, be_h1, ge_ln2, be_ln2, We_h2, be_h2, We_out,
                     be_out)
    final = _run_combine(y, pos_a, pos_b, w_a, w_b)
    return (final, lb.reshape(()), gate_probs)


# speedup vs baseline: 1.2271x; 1.2271x over previous
import functools

import jax
import jax.numpy as jnp
from jax import lax
from jax.experimental import pallas as pl
from jax.experimental.pallas import tpu as pltpu
from jax.experimental.pallas import tpu_sc as plsc

T, D, H, O, E, TOPK = 2048, 1024, 1024, 1024, 8, 2
GH = 256
K2 = H // 2
BT = 256
NA = T * TOPK
NBLK = NA // BT + (E - 1)
NPAD = NBLK * BT
NW = 32
LNEPS = 1e-5


def _ln(x, g, b):
    mu = jnp.mean(x, axis=-1, keepdims=True)
    var = jnp.mean((x - mu) ** 2, axis=-1, keepdims=True)
    return (x - mu) * jax.lax.rsqrt(var + LNEPS) * g + b


def _gating_body(x_ref, wgin_ref, bgin_ref, ggln1_ref, bgln1_ref, wgh1_ref,
                 bgh1_ref, ggln2_ref, bgln2_ref, wgh2_ref, bgh2_ref,
                 wgout_ref, bgout_ref, temp_ref,
                 probs_ref, wa_ref, wb_ref, pos_ref, counts_ref, lb_ref):
    x = x_ref[...]
    h0 = jax.nn.relu(
        lax.dot_general(x, wgin_ref[...], (((1,), (1,)), ((), ())),
                        preferred_element_type=jnp.float32) + bgin_ref[...])
    h1 = jax.nn.relu(_ln(h0, ggln1_ref[...], bgln1_ref[...]))
    h1 = lax.dot_general(h1, wgh1_ref[...], (((1,), (1,)), ((), ())),
                         preferred_element_type=jnp.float32) + bgh1_ref[...]
    h1 = h1 + h0
    h2 = jax.nn.relu(_ln(h1, ggln2_ref[...], bgln2_ref[...]))
    h2 = lax.dot_general(h2, wgh2_ref[...], (((1,), (1,)), ((), ())),
                         preferred_element_type=jnp.float32) + bgh2_ref[...]
    logits = lax.dot_general(h2, wgout_ref[...], (((1,), (1,)), ((), ())),
                             preferred_element_type=jnp.float32) + bgout_ref[...]
    temp = jnp.clip(temp_ref[0, 0], 0.5, 5.0)
    logits = logits / temp
    logits = logits - jnp.max(logits, axis=1, keepdims=True)
    ex = jnp.exp(logits)
    p = ex / jnp.sum(ex, axis=1, keepdims=True)
    probs_ref[...] = p

    usage = jnp.mean(p, axis=0, keepdims=True)
    lb_ref[...] = jnp.sum((usage - 1.0 / E) ** 2).reshape(1, 1) * 0.05

    iota = lax.broadcasted_iota(jnp.int32, (T, E), 1).astype(jnp.float32)
    m1 = jnp.max(p, axis=1, keepdims=True)
    i1 = jnp.min(jnp.where(p == m1, iota, float(E)), axis=1, keepdims=True)
    pm = jnp.where(iota == i1, -jnp.inf, p)
    m2 = jnp.max(pm, axis=1, keepdims=True)
    i2 = jnp.min(jnp.where(pm == m2, iota, float(E)), axis=1, keepdims=True)
    ssum = m1 + m2
    wa_ref[...] = jnp.broadcast_to(m1 / ssum, (T, 16))
    wb_ref[...] = jnp.broadcast_to(m2 / ssum, (T, 16))

    oh = jnp.concatenate([(iota == i1).astype(jnp.float32),
                          (iota == i2).astype(jnp.float32)], axis=0)

    C, Rr = 8, NA // 8
    tri = (lax.broadcasted_iota(jnp.int32, (Rr, Rr), 0)
           >= lax.broadcasted_iota(jnp.int32, (Rr, Rr), 1)).astype(jnp.float32)
    offs = jnp.zeros((1, E), jnp.float32)
    pos_rows = []
    for c in range(C):
        ohc = oh[c * Rr:(c + 1) * Rr]
        within = lax.dot_general(tri, ohc, (((1,), (0,)), ((), ())),
                                 preferred_element_type=jnp.float32)
        rank_c = (jnp.sum(within * ohc, axis=1, keepdims=True) - 1.0
                  + jnp.sum(ohc * offs, axis=1, keepdims=True))
        pos_rows.append((rank_c, ohc))
        offs = offs + within[Rr - 1:Rr, :]
    counts = offs
    counts_ref[...] = counts.astype(jnp.int32)

    sizes_al = jnp.floor((counts + (BT - 1)) / BT) * BT
    tri8 = (lax.broadcasted_iota(jnp.int32, (E, E), 0)
            < lax.broadcasted_iota(jnp.int32, (E, E), 1)).astype(jnp.float32)
    starts = lax.dot_general(sizes_al, tri8, (((1,), (0,)), ((), ())),
                             preferred_element_type=jnp.float32)
    for c in range(C):
        rank_c, ohc = pos_rows[c]
        pos_c = rank_c + jnp.sum(ohc * starts, axis=1, keepdims=True)
        pos_ref[c:c + 1, :] = pos_c.reshape(1, Rr).astype(jnp.int32)


def _run_gating(x, W_gin, b_gin, g_gln1, b_gln1, W_gh1, b_gh1, g_gln2, b_gln2,
                W_gh2, b_gh2, W_gout, b_gout, temperature):
    out_shapes = (
        jax.ShapeDtypeStruct((T, E), jnp.float32),
        jax.ShapeDtypeStruct((T, 16), jnp.float32),
        jax.ShapeDtypeStruct((T, 16), jnp.float32),
        jax.ShapeDtypeStruct((8, NA // 8), jnp.int32),
        jax.ShapeDtypeStruct((1, E), jnp.int32),
        jax.ShapeDtypeStruct((1, 1), jnp.float32),
    )
    return pl.pallas_call(
        _gating_body,
        out_shape=out_shapes,
    )(x, W_gin, b_gin.reshape(1, GH), g_gln1.reshape(1, GH),
      b_gln1.reshape(1, GH), W_gh1, b_gh1.reshape(1, GH),
      g_gln2.reshape(1, GH), b_gln2.reshape(1, GH), W_gh2,
      b_gh2.reshape(1, 128), W_gout, b_gout.reshape(1, E),
      temperature.reshape(1, 1))


def _dispatch_body(x_hbm, pa_hbm, pb_hbm, xs_hbm, rows_v, ia_v, ib_v,
                   sema, semb):
    wid = lax.axis_index("s") * 2 + lax.axis_index("c")
    per = T // NW
    for it in range(per // 32):
        tb = wid * per + it * 32
        pltpu.sync_copy(pa_hbm.at[pl.ds(tb, 32)], ia_v)
        pltpu.sync_copy(pb_hbm.at[pl.ds(tb, 32)], ib_v)
        pltpu.sync_copy(x_hbm.at[pl.ds(tb, 32)], rows_v)
        ca = pltpu.async_copy(rows_v, xs_hbm.at[ia_v], sema)
        cb = pltpu.async_copy(rows_v, xs_hbm.at[ib_v], semb)
        ca.wait()
        cb.wait()


def _run_dispatch(x, pos_a, pos_b):
    mesh = plsc.VectorSubcoreMesh(core_axis_name="c", subcore_axis_name="s")
    f = pl.kernel(
        _dispatch_body,
        out_type=jax.ShapeDtypeStruct((NPAD, D), jnp.float32),
        mesh=mesh,
        scratch_types=[
            pltpu.VMEM((32, D), jnp.float32),
            pltpu.VMEM((32,), jnp.int32),
            pltpu.VMEM((32,), jnp.int32),
            pltpu.SemaphoreType.DMA,
            pltpu.SemaphoreType.DMA,
        ],
    )
    return f(x, pos_a, pos_b)


def _expert_body(be_ref, xs_ref, win_ref, wh1_ref, wh2_ref, wout_ref,
                 vec_ref, y_ref):
    xb = xs_ref[...]
    bin_ = vec_ref[0, 0:1, :]
    gln1 = vec_ref[0, 1:2, :]
    bln1 = vec_ref[0, 2:3, :]
    bh1 = vec_ref[0, 3:4, :]
    gln2 = vec_ref[0, 4:5, :]
    bln2 = vec_ref[0, 5:6, :]
    bh2 = vec_ref[0, 6:7, :K2]
    bout = vec_ref[0, 7:8, :]
    h0 = jax.nn.relu(
        lax.dot_general(xb, win_ref[0], (((1,), (1,)), ((), ())),
                        preferred_element_type=jnp.float32) + bin_)
    t1 = jax.nn.relu(_ln(h0, gln1, bln1))
    t1 = lax.dot_general(t1, wh1_ref[0], (((1,), (1,)), ((), ())),
                         preferred_element_type=jnp.float32) + bh1
    t1 = t1 + h0
    t2 = _ln(t1, gln2, bln2)
    t2 = t2 * jax.nn.sigmoid(t2)
    t2 = lax.dot_general(t2, wh2_ref[0], (((1,), (1,)), ((), ())),
                         preferred_element_type=jnp.float32) + bh2
    y_ref[...] = lax.dot_general(t2, wout_ref[0],
                                 (((1,), (1,)), ((), ())),
                                 preferred_element_type=jnp.float32) + bout


def _run_experts(x_sorted, blk_expert, We_in, be_in, ge_ln1, be_ln1, We_h1,
                 be_h1, ge_ln2, be_ln2, We_h2, be_h2, We_out, be_out):
    def im_x(b, be):
        return (b, 0)

    def im_e3(b, be):
        return (be[b], 0, 0)

    def im_e2(b, be):
        return (be[b], 0)

    grid_spec = pltpu.PrefetchScalarGridSpec(
        num_scalar_prefetch=1,
        grid=(NBLK,),
        in_specs=[
            pl.BlockSpec((BT, D), im_x),
            pl.BlockSpec((1, H, D), im_e3),
            pl.BlockSpec((1, H, H), im_e3),
            pl.BlockSpec((1, K2, H), im_e3),
            pl.BlockSpec((1, O, K2), im_e3),
            pl.BlockSpec((1, 8, H), im_e3),
        ],
        out_specs=pl.BlockSpec((BT, O), im_x),
    )
    call = pl.pallas_call(
        _expert_body,
        grid_spec=grid_spec,
        out_shape=jax.ShapeDtypeStruct((NPAD, O), jnp.float32),
        compiler_params=pltpu.CompilerParams(
            dimension_semantics=("arbitrary",)),
    )
    vec_pack = jnp.stack(
        [be_in, ge_ln1, be_ln1, be_h1, ge_ln2, be_ln2,
         jnp.pad(be_h2, ((0, 0), (0, H - K2))), be_out], axis=1)
    return call(blk_expert, x_sorted, We_in, We_h1, We_h2, We_out, vec_pack)


def _combine_body(y_hbm, pa_hbm, pb_hbm, wa_hbm, wb_hbm, fin_hbm,
                  bufa, bufb, ia_v, ib_v, wa_v, wb_v, sema, semb):
    wid = lax.axis_index("s") * 2 + lax.axis_index("c")
    per = T // NW
    for it in range(per // 32):
        tb = wid * per + it * 32
        pltpu.sync_copy(pa_hbm.at[pl.ds(tb, 32)], ia_v)
        pltpu.sync_copy(pb_hbm.at[pl.ds(tb, 32)], ib_v)
        ca = pltpu.async_copy(y_hbm.at[ia_v], bufa, sema)
        cb = pltpu.async_copy(y_hbm.at[ib_v], bufb, semb)
        pltpu.sync_copy(wa_hbm.at[pl.ds(tb, 32)], wa_v)
        pltpu.sync_copy(wb_hbm.at[pl.ds(tb, 32)], wb_v)
        ca.wait()
        cb.wait()

        def body(i, carry):
            va = wa_v[i]
            vb = wb_v[i]
            for j in range(O // 16):
                sl = pl.ds(j * 16, 16)
                bufa[i, sl] = va * bufa[i, sl] + vb * bufb[i, sl]
            return carry

        lax.fori_loop(0, 32, body, 0)
        pltpu.sync_copy(bufa, fin_hbm.at[pl.ds(tb, 32)])


def _run_combine(y, pos_a, pos_b, w_a, w_b):
    mesh = plsc.VectorSubcoreMesh(core_axis_name="c", subcore_axis_name="s")
    f = pl.kernel(
        _combine_body,
        out_type=jax.ShapeDtypeStruct((T, O), jnp.float32),
        mesh=mesh,
        scratch_types=[
            pltpu.VMEM((32, O), jnp.float32),
            pltpu.VMEM((32, O), jnp.float32),
            pltpu.VMEM((32,), jnp.int32),
            pltpu.VMEM((32,), jnp.int32),
            pltpu.VMEM((32, 16), jnp.float32),
            pltpu.VMEM((32, 16), jnp.float32),
            pltpu.SemaphoreType.DMA,
            pltpu.SemaphoreType.DMA,
        ],
    )
    return f(y, pos_a, pos_b, w_a, w_b)


def kernel(x, W_gin, b_gin, g_gln1, b_gln1, W_gh1, b_gh1, g_gln2, b_gln2,
           W_gh2, b_gh2, W_gout, b_gout, temperature, We_in, be_in, ge_ln1,
           be_ln1, We_h1, be_h1, ge_ln2, be_ln2, We_h2, be_h2, We_out,
           be_out):
    gate_probs, w_a, w_b, pos8, counts, lb = _run_gating(
        x, W_gin, b_gin, g_gln1, b_gln1, W_gh1, b_gh1, g_gln2, b_gln2,
        W_gh2, b_gh2, W_gout, b_gout, temperature)

    pos_flat = pos8.reshape(NA)
    pos_a = pos_flat[:T]
    pos_b = pos_flat[T:]

    counts_i = counts.reshape(E)
    sizes_al = ((counts_i + (BT - 1)) // BT) * BT
    starts = jnp.concatenate(
        [jnp.zeros((1,), jnp.int32), jnp.cumsum(sizes_al)[:-1]]).astype(jnp.int32)
    b_lo = jnp.arange(NBLK, dtype=jnp.int32) * BT
    in_blk = (b_lo[:, None] >= starts[None, :]) & \
             (b_lo[:, None] < (starts + sizes_al)[None, :])
    blk_expert = jnp.sum(
        in_blk * jnp.arange(E, dtype=jnp.int32)[None, :], axis=1).astype(jnp.int32)

    x_sorted = _run_dispatch(x, pos_a, pos_b)
    y = _run_experts(x_sorted, blk_expert, We_in, be_in, ge_ln1, be_ln1,
                     We_h1, be_h1, ge_ln2, be_ln2, We_h2, be_h2, We_out,
                     be_out)
    final = _run_combine(y, pos_a, pos_b, w_a, w_b)
    return (final, lb.reshape(()), gate_probs)
